# baseline (speedup 1.0000x reference)
import functools

import jax
import jax.numpy as jnp
from jax import lax
from jax.experimental import pallas as pl
from jax.experimental.pallas import tpu as pltpu
from jax.experimental.pallas import tpu_sc as plsc

_TOKENS = 32768
_EXPERTS = 64
_NW = 32
_TPW = _TOKENS // _NW


def _sc_body(x_hbm, out_hbm, x_v, st_v, part_v):
    nc = 2
    wid = lax.axis_index("s") * nc + lax.axis_index("c")
    base = wid * (_TPW * _EXPERTS)
    pltpu.sync_copy(x_hbm.at[pl.ds(base, _TPW * _EXPERTS)], x_v)

    ia = jnp.arange(16, dtype=jnp.int32)
    idx_a = jnp.where(ia < 8, ia + 8, ia + 16)
    idx_b = idx_a + 32
    idx_t8 = jnp.full((16,), 8, dtype=jnp.int32)
    idx_mx = jnp.full((16,), 15, dtype=jnp.int32)
    one = jnp.ones((16,), dtype=jnp.float32)
    zero = jnp.zeros((16,), dtype=jnp.float32)

    def body(t, carry):
        p0, p1, p2, p3, m0, m1, m2, m3 = carry
        off = t * _EXPERTS
        v0 = x_v[pl.ds(off, 16)]
        v1 = x_v[pl.ds(off + 16, 16)]
        v2 = x_v[pl.ds(off + 32, 16)]
        v3 = x_v[pl.ds(off + 48, 16)]

        st_v[pl.ds(0, 16)] = jnp.sort(v0)
        st_v[pl.ds(16, 16)] = jnp.sort(v1)
        st_v[pl.ds(32, 16)] = jnp.sort(v2)
        st_v[pl.ds(48, 16)] = jnp.sort(v3)
        c01 = plsc.load_gather(st_v, [idx_a])
        c23 = plsc.load_gather(st_v, [idx_b])
        st_v[pl.ds(0, 16)] = jnp.sort(c01)
        st_v[pl.ds(16, 16)] = jnp.sort(c23)
        cf = plsc.load_gather(st_v, [idx_a])
        st_v[pl.ds(0, 16)] = jnp.sort(cf)
        t8 = plsc.load_gather(st_v, [idx_t8])
        mx = plsc.load_gather(st_v, [idx_mx])

        m0 = jnp.maximum(m0, jnp.where(v0 >= t8, one, zero))
        m1 = jnp.maximum(m1, jnp.where(v1 >= t8, one, zero))
        m2 = jnp.maximum(m2, jnp.where(v2 >= t8, one, zero))
        m3 = jnp.maximum(m3, jnp.where(v3 >= t8, one, zero))

        e0 = jnp.exp(v0 - mx)
        e1 = jnp.exp(v1 - mx)
        e2 = jnp.exp(v2 - mx)
        e3 = jnp.exp(v3 - mx)
        tot = (e0 + e1) + (e2 + e3)
        sv = jnp.broadcast_to(jnp.sum(tot), (16,))
        r = lax.bitcast_convert_type(
            jnp.int32(0x7EF311C3) - lax.bitcast_convert_type(sv, jnp.int32),
            jnp.float32,
        )
        two = jnp.float32(2.0)
        r = r * (two - sv * r)
        r = r * (two - sv * r)
        inv = r * (two - sv * r)
        p0 = p0 + e0 * inv
        p1 = p1 + e1 * inv
        p2 = p2 + e2 * inv
        p3 = p3 + e3 * inv
        return p0, p1, p2, p3, m0, m1, m2, m3

    init = (zero, zero, zero, zero, zero, zero, zero, zero)
    p0, p1, p2, p3, m0, m1, m2, m3 = lax.fori_loop(0, _TPW, body, init)

    part_v[pl.ds(0, 16)] = p0
    part_v[pl.ds(16, 16)] = p1
    part_v[pl.ds(32, 16)] = p2
    part_v[pl.ds(48, 16)] = p3
    part_v[pl.ds(64, 16)] = m0
    part_v[pl.ds(80, 16)] = m1
    part_v[pl.ds(96, 16)] = m2
    part_v[pl.ds(112, 16)] = m3
    pltpu.sync_copy(part_v, out_hbm.at[pl.ds(wid * 128, 128)])


def _tc_final(x_ref, out_ref):
    x = x_ref[...]
    psum = jnp.sum(x[:, :_EXPERTS], axis=0)
    mask = jnp.max(x[:, _EXPERTS:], axis=0)
    t = jnp.sum(psum) * jnp.float32(1.0 / _TOKENS)
    msum = jnp.sum(mask)
    out_ref[...] = jnp.full((1, 1), msum * t, jnp.float32)


def kernel(gate_logits):
    mesh = plsc.VectorSubcoreMesh(core_axis_name="c", subcore_axis_name="s")
    sc = pl.kernel(
        _sc_body,
        mesh=mesh,
        out_type=jax.ShapeDtypeStruct((_NW * 128,), jnp.float32),
        scratch_types=[
            pltpu.VMEM((_TPW * _EXPERTS,), jnp.float32),
            pltpu.VMEM((64,), jnp.float32),
            pltpu.VMEM((128,), jnp.float32),
        ],
        compiler_params=pltpu.CompilerParams(needs_layout_passes=False),
    )
    partials = sc(gate_logits.reshape(-1)).reshape(_NW, 128)
    out = pl.pallas_call(
        _tc_final,
        out_shape=jax.ShapeDtypeStruct((1, 1), jnp.float32),
    )(partials)
    return out[0, 0]


# speedup vs baseline: 1.4774x; 1.4774x over previous
import functools

import jax
import jax.numpy as jnp
from jax import lax
from jax.experimental import pallas as pl
from jax.experimental.pallas import tpu as pltpu
from jax.experimental.pallas import tpu_sc as plsc

_TOKENS = 32768
_EXPERTS = 64
_NW = 32
_TPW = _TOKENS // _NW


def _sc_body(x_hbm, out_hbm, x_v, part_v):
    nc = 2
    wid = lax.axis_index("s") * nc + lax.axis_index("c")
    base = wid * (_TPW * _EXPERTS)
    pltpu.sync_copy(x_hbm.at[pl.ds(base, _TPW * _EXPERTS)], x_v)

    ia = jnp.arange(16, dtype=jnp.int32)
    lt8 = ia < 8
    idx_t8 = jnp.full((16,), 8, dtype=jnp.int32)
    idx_mx = jnp.full((16,), 15, dtype=jnp.int32)
    one = jnp.ones((16,), dtype=jnp.float32)
    zero = jnp.zeros((16,), dtype=jnp.float32)
    neg = jnp.full((16,), -jnp.inf, dtype=jnp.float32)

    def _rev(x):
        return lax.rev(x, (0,))

    init = (zero, zero, zero, zero, neg, neg, neg, neg)

    @plsc.parallel_loop(0, _TPW, unroll=4, carry=init)
    def body(t, carry):
        p0, p1, p2, p3, m0, m1, m2, m3 = carry
        off = t * _EXPERTS
        v0 = x_v[pl.ds(off, 16)]
        v1 = x_v[pl.ds(off + 16, 16)]
        v2 = x_v[pl.ds(off + 32, 16)]
        v3 = x_v[pl.ds(off + 48, 16)]

        s0 = jnp.sort(v0)
        s1 = jnp.sort(v1)
        s2 = jnp.sort(v2)
        s3 = jnp.sort(v3)
        c01 = jnp.where(lt8, _rev(s0), s1)
        c23 = jnp.where(lt8, _rev(s2), s3)
        cf = jnp.where(lt8, _rev(jnp.sort(c01)), jnp.sort(c23))
        sf = jnp.sort(cf)
        t8 = sf.at[idx_t8].get(mode="promise_in_bounds")
        mx = sf.at[idx_mx].get(mode="promise_in_bounds")

        m0 = jnp.maximum(m0, v0 - t8)
        m1 = jnp.maximum(m1, v1 - t8)
        m2 = jnp.maximum(m2, v2 - t8)
        m3 = jnp.maximum(m3, v3 - t8)

        e0 = jnp.exp(v0 - mx)
        e1 = jnp.exp(v1 - mx)
        e2 = jnp.exp(v2 - mx)
        e3 = jnp.exp(v3 - mx)
        tot = (e0 + e1) + (e2 + e3)
        sv = jnp.broadcast_to(jnp.sum(tot), (16,))
        r = lax.bitcast_convert_type(
            jnp.int32(0x7EF311C3) - lax.bitcast_convert_type(sv, jnp.int32),
            jnp.float32,
        )
        two = jnp.float32(2.0)
        r = r * (two - sv * r)
        inv = r * (two - sv * r)
        p0 = p0 + e0 * inv
        p1 = p1 + e1 * inv
        p2 = p2 + e2 * inv
        p3 = p3 + e3 * inv
        return p0, p1, p2, p3, m0, m1, m2, m3

    p0, p1, p2, p3, m0, m1, m2, m3 = body
    m0 = jnp.where(m0 >= 0.0, one, zero)
    m1 = jnp.where(m1 >= 0.0, one, zero)
    m2 = jnp.where(m2 >= 0.0, one, zero)
    m3 = jnp.where(m3 >= 0.0, one, zero)

    part_v[pl.ds(0, 16)] = p0
    part_v[pl.ds(16, 16)] = p1
    part_v[pl.ds(32, 16)] = p2
    part_v[pl.ds(48, 16)] = p3
    part_v[pl.ds(64, 16)] = m0
    part_v[pl.ds(80, 16)] = m1
    part_v[pl.ds(96, 16)] = m2
    part_v[pl.ds(112, 16)] = m3
    pltpu.sync_copy(part_v, out_hbm.at[pl.ds(wid * 128, 128)])


def _tc_final(x_ref, out_ref):
    x = x_ref[...]
    psum = jnp.sum(x[:, :_EXPERTS], axis=0)
    mask = jnp.max(x[:, _EXPERTS:], axis=0)
    t = jnp.sum(psum) * jnp.float32(1.0 / _TOKENS)
    msum = jnp.sum(mask)
    out_ref[...] = jnp.full((1, 1), msum * t, jnp.float32)


def kernel(gate_logits):
    mesh = plsc.VectorSubcoreMesh(core_axis_name="c", subcore_axis_name="s")
    sc = pl.kernel(
        _sc_body,
        mesh=mesh,
        out_type=jax.ShapeDtypeStruct((_NW * 128,), jnp.float32),
        scratch_types=[
            pltpu.VMEM((_TPW * _EXPERTS,), jnp.float32),
            pltpu.VMEM((128,), jnp.float32),
        ],
        compiler_params=pltpu.CompilerParams(needs_layout_passes=False),
    )
    partials = sc(gate_logits.reshape(-1)).reshape(_NW, 128)
    out = pl.pallas_call(
        _tc_final,
        out_shape=jax.ShapeDtypeStruct((1, 1), jnp.float32),
    )(partials)
    return out[0, 0]


# speedup vs baseline: 1.7453x; 1.1814x over previous
import functools

import jax
import jax.numpy as jnp
from jax import lax
from jax.experimental import pallas as pl
from jax.experimental.pallas import tpu as pltpu
from jax.experimental.pallas import tpu_sc as plsc

_TOKENS = 32768
_EXPERTS = 64
_NW = 32
_TPW = _TOKENS // _NW


def _sc_body(x_hbm, out_hbm, x_v, part_v):
    nc = 2
    wid = lax.axis_index("s") * nc + lax.axis_index("c")

    ia = jnp.arange(16, dtype=jnp.int32)
    lt8 = ia < 8
    idx_t8 = jnp.full((16,), 8, dtype=jnp.int32)
    idx_mx = jnp.full((16,), 15, dtype=jnp.int32)
    one = jnp.ones((16,), dtype=jnp.float32)
    zero = jnp.zeros((16,), dtype=jnp.float32)
    neg = jnp.full((16,), -jnp.inf, dtype=jnp.float32)

    def _rev(x):
        return lax.rev(x, (0,))

    acc = (zero, zero, zero, zero, neg, neg, neg, neg)
    chunk = _TPW // 2
    for c in range(2):
        pltpu.sync_copy(x_hbm.at[pl.ds(wid * _TPW + c * chunk, chunk)], x_v)

        @plsc.parallel_loop(0, chunk, unroll=4, carry=acc)
        def body(t, carry):
            p0, p1, p2, p3, m0, m1, m2, m3 = carry
            v0 = x_v[t, pl.ds(0, 16)]
            v1 = x_v[t, pl.ds(16, 16)]
            v2 = x_v[t, pl.ds(32, 16)]
            v3 = x_v[t, pl.ds(48, 16)]

            s0 = jnp.sort(v0)
            s1 = jnp.sort(v1)
            s2 = jnp.sort(v2)
            s3 = jnp.sort(v3)
            c01 = jnp.where(lt8, _rev(s0), s1)
            c23 = jnp.where(lt8, _rev(s2), s3)
            cf = jnp.where(lt8, _rev(jnp.sort(c01)), jnp.sort(c23))
            sf = jnp.sort(cf)
            t8 = sf.at[idx_t8].get(mode="promise_in_bounds")
            mx = sf.at[idx_mx].get(mode="promise_in_bounds")

            m0 = jnp.maximum(m0, v0 - t8)
            m1 = jnp.maximum(m1, v1 - t8)
            m2 = jnp.maximum(m2, v2 - t8)
            m3 = jnp.maximum(m3, v3 - t8)

            e0 = jnp.exp(v0 - mx)
            e1 = jnp.exp(v1 - mx)
            e2 = jnp.exp(v2 - mx)
            e3 = jnp.exp(v3 - mx)
            tot = (e0 + e1) + (e2 + e3)
            sv = jnp.broadcast_to(jnp.sum(tot), (16,))
            r = lax.bitcast_convert_type(
                jnp.int32(0x7EF311C3) - lax.bitcast_convert_type(sv, jnp.int32),
                jnp.float32,
            )
            two = jnp.float32(2.0)
            r = r * (two - sv * r)
            inv = r * (two - sv * r)
            p0 = p0 + e0 * inv
            p1 = p1 + e1 * inv
            p2 = p2 + e2 * inv
            p3 = p3 + e3 * inv
            return p0, p1, p2, p3, m0, m1, m2, m3

        acc = body

    p0, p1, p2, p3, m0, m1, m2, m3 = acc
    m0 = jnp.where(m0 >= 0.0, one, zero)
    m1 = jnp.where(m1 >= 0.0, one, zero)
    m2 = jnp.where(m2 >= 0.0, one, zero)
    m3 = jnp.where(m3 >= 0.0, one, zero)

    part_v[pl.ds(0, 16)] = p0
    part_v[pl.ds(16, 16)] = p1
    part_v[pl.ds(32, 16)] = p2
    part_v[pl.ds(48, 16)] = p3
    part_v[pl.ds(64, 16)] = m0
    part_v[pl.ds(80, 16)] = m1
    part_v[pl.ds(96, 16)] = m2
    part_v[pl.ds(112, 16)] = m3
    pltpu.sync_copy(part_v, out_hbm.at[pl.ds(wid * 128, 128)])


def _tc_final(x_ref, out_ref):
    x = x_ref[...]
    psum = jnp.sum(x[:, :_EXPERTS], axis=0)
    mask = jnp.max(x[:, _EXPERTS:], axis=0)
    t = jnp.sum(psum) * jnp.float32(1.0 / _TOKENS)
    msum = jnp.sum(mask)
    out_ref[...] = jnp.full((1, 1), msum * t, jnp.float32)


def kernel(gate_logits):
    mesh = plsc.VectorSubcoreMesh(core_axis_name="c", subcore_axis_name="s")
    sc = pl.kernel(
        _sc_body,
        mesh=mesh,
        out_type=jax.ShapeDtypeStruct((_NW * 128,), jnp.float32),
        scratch_types=[
            pltpu.VMEM((_TPW // 2, _EXPERTS), jnp.float32),
            pltpu.VMEM((128,), jnp.float32),
        ],
        compiler_params=pltpu.CompilerParams(needs_layout_passes=False),
    )
    partials = sc(gate_logits).reshape(_NW, 128)
    out = pl.pallas_call(
        _tc_final,
        out_shape=jax.ShapeDtypeStruct((1, 1), jnp.float32),
    )(partials)
    return out[0, 0]


# speedup vs baseline: 1.7472x; 1.0011x over previous
import functools

import jax
import jax.numpy as jnp
from jax import lax
from jax.experimental import pallas as pl
from jax.experimental.pallas import tpu as pltpu
from jax.experimental.pallas import tpu_sc as plsc

_TOKENS = 32768
_EXPERTS = 64
_NW = 32
_TPW = _TOKENS // _NW


def _sc_body(x_hbm, out_hbm, x_v, part_v):
    nc = 2
    wid = lax.axis_index("s") * nc + lax.axis_index("c")

    ia = jnp.arange(16, dtype=jnp.int32)
    lt8 = ia < 8
    idx_t8 = jnp.full((16,), 8, dtype=jnp.int32)
    idx_mx = jnp.full((16,), 15, dtype=jnp.int32)
    one = jnp.ones((16,), dtype=jnp.float32)
    zero = jnp.zeros((16,), dtype=jnp.float32)
    neg = jnp.full((16,), -jnp.inf, dtype=jnp.float32)

    def _rev(x):
        return lax.rev(x, (0,))

    acc = (zero, zero, zero, zero, neg, neg, neg, neg)
    chunk = _TPW // 2
    for c in range(2):
        pltpu.sync_copy(x_hbm.at[pl.ds(wid * _TPW + c * chunk, chunk)], x_v)

        @plsc.parallel_loop(0, chunk, unroll=4, carry=acc)
        def body(t, carry):
            p0, p1, p2, p3, m0, m1, m2, m3 = carry
            v0 = x_v[t, pl.ds(0, 16)]
            v1 = x_v[t, pl.ds(16, 16)]
            v2 = x_v[t, pl.ds(32, 16)]
            v3 = x_v[t, pl.ds(48, 16)]

            s0 = jnp.sort(v0)
            s1 = jnp.sort(v1)
            s2 = jnp.sort(v2)
            s3 = jnp.sort(v3)
            c01 = jnp.where(lt8, _rev(s0), s1)
            c23 = jnp.where(lt8, _rev(s2), s3)
            cf = jnp.where(lt8, _rev(jnp.sort(c01)), jnp.sort(c23))
            sf = jnp.sort(cf)
            t8 = sf.at[idx_t8].get(mode="promise_in_bounds")
            mx = sf.at[idx_mx].get(mode="promise_in_bounds")

            m0 = jnp.maximum(m0, v0 - t8)
            m1 = jnp.maximum(m1, v1 - t8)
            m2 = jnp.maximum(m2, v2 - t8)
            m3 = jnp.maximum(m3, v3 - t8)

            e0 = jnp.exp(v0 - mx)
            e1 = jnp.exp(v1 - mx)
            e2 = jnp.exp(v2 - mx)
            e3 = jnp.exp(v3 - mx)
            tot = (e0 + e1) + (e2 + e3)
            sv = jnp.broadcast_to(jnp.sum(tot), (16,))
            r = lax.bitcast_convert_type(
                jnp.int32(0x7EF311C3) - lax.bitcast_convert_type(sv, jnp.int32),
                jnp.float32,
            )
            two = jnp.float32(2.0)
            r = r * (two - sv * r)
            inv = r * (two - sv * r)
            p0 = p0 + e0 * inv
            p1 = p1 + e1 * inv
            p2 = p2 + e2 * inv
            p3 = p3 + e3 * inv
            return p0, p1, p2, p3, m0, m1, m2, m3

        acc = body

    p0, p1, p2, p3, m0, m1, m2, m3 = acc
    m0 = jnp.where(m0 >= 0.0, one, zero)
    m1 = jnp.where(m1 >= 0.0, one, zero)
    m2 = jnp.where(m2 >= 0.0, one, zero)
    m3 = jnp.where(m3 >= 0.0, one, zero)

    part_v[pl.ds(0, 16)] = p0
    part_v[pl.ds(16, 16)] = p1
    part_v[pl.ds(32, 16)] = p2
    part_v[pl.ds(48, 16)] = p3
    part_v[pl.ds(64, 16)] = m0
    part_v[pl.ds(80, 16)] = m1
    part_v[pl.ds(96, 16)] = m2
    part_v[pl.ds(112, 16)] = m3
    pltpu.sync_copy(part_v, out_hbm.at[pl.ds(wid * 128, 128)])


def _tc_final(x_ref, out_ref):
    x = x_ref[...]
    psum = jnp.sum(x[:, :_EXPERTS], axis=0)
    mask = jnp.max(x[:, _EXPERTS:], axis=0)
    t = jnp.sum(psum) * jnp.float32(1.0 / _TOKENS)
    msum = jnp.sum(mask)
    out_ref[...] = jnp.full((1, 1), msum * t, jnp.float32)


def kernel(gate_logits):
    mesh = plsc.VectorSubcoreMesh(core_axis_name="c", subcore_axis_name="s")
    sc = pl.kernel(
        _sc_body,
        mesh=mesh,
        out_type=jax.ShapeDtypeStruct((_NW * 128,), jnp.float32),
        scratch_types=[
            pltpu.VMEM((_TPW // 2, _EXPERTS), jnp.float32),
            pltpu.VMEM((128,), jnp.float32),
        ],
        compiler_params=pltpu.CompilerParams(
            needs_layout_passes=False, use_tc_tiling_on_sc=True
        ),
    )
    partials = sc(gate_logits).reshape(_NW, 128)
    out = pl.pallas_call(
        _tc_final,
        out_shape=jax.ShapeDtypeStruct((1, 1), jnp.float32),
    )(partials)
    return out[0, 0]
